# baseline (speedup 1.0000x reference)
import functools

import jax
import jax.numpy as jnp
from jax import lax
from jax.experimental import pallas as pl
from jax.experimental.pallas import tpu as pltpu

CAPACITY_FACTOR = 1.1
TOK_BLK = 512


def _select_and_emit(ep, psum, disp_ref, comb_ref, aux_ref, *,
                     n_tokens, n_experts, capacity):
    f32 = jnp.float32
    bf16 = jnp.bfloat16
    T = n_tokens
    T2 = 2 * n_tokens
    E = n_experts
    nb = 64

    e_all = jnp.concatenate([ep[0:1, :], ep[1:2, :]], axis=1).astype(jnp.int32)
    p_all = jnp.concatenate([ep[2:3, :], ep[3:4, :]], axis=1)
    hi = lax.bitcast_convert_type(p_all, jnp.int32)
    pos = lax.broadcasted_iota(jnp.int32, (1, T2), 1)
    flat = jnp.where(pos < T, 2 * pos, 2 * (pos - T) + 1)
    lo = (T2 - 1) - flat

    erow = lax.broadcasted_iota(jnp.int32, (E, T2), 0)
    one_b = jnp.ones((), bf16)
    zb = jnp.zeros((), bf16)
    erow_b = erow.astype(bf16)
    e_all_b = e_all.astype(bf16)
    oh = jnp.where(erow_b == e_all_b, one_b, zb)

    kept = jnp.zeros((1, T2), f32)
    act = jnp.ones((1, T2), bf16)
    need = jnp.full((1, E), capacity, f32)

    bv = lax.broadcasted_iota(jnp.int32, (nb, nb), 0)
    bu = lax.broadcasted_iota(jnp.int32, (nb, nb), 1)
    cumM = (bu >= bv).astype(f32)
    vio0 = lax.broadcasted_iota(jnp.int32, (nb, E), 0)
    drow_b = lax.broadcasted_iota(jnp.int32, (nb, T2), 0).astype(bf16)

    counts = None
    passes = [(hi, sh) for sh in (24, 18, 12, 6, 0)] + \
             [(lo, sh) for sh in (12, 6, 0)]
    n_pass = len(passes)
    for pi, (w_, sh) in enumerate(passes):
        d = (w_ >> sh) & (nb - 1)
        df = d.astype(f32)
        d_b = d.astype(bf16)
        Dp = jnp.where(drow_b == d_b, act, zb)
        hist = lax.dot_general(Dp, oh, (((1,), (1,)), ((), ())),
                               preferred_element_type=f32)
        cum = lax.dot_general(cumM, hist, (((1,), (0,)), ((), ())),
                              preferred_element_type=f32)
        if pi == 0:
            counts = cum[0:1, :]
        t_row = jnp.max(jnp.where(cum >= need, vio0, -1), axis=0,
                        keepdims=True)
        cnt = jnp.sum(jnp.where(vio0 == t_row, cum - hist, 0.0), axis=0,
                      keepdims=True)
        cnt = jnp.where(t_row == -1, cum[0:1, :], cnt)
        need = need - cnt
        tg = lax.dot_general(t_row.astype(bf16), oh, (((1,), (0,)), ((), ())),
                             preferred_element_type=f32)
        if pi == n_pass - 1:
            kept = kept + act.astype(f32) * (df >= tg).astype(f32)
        else:
            kept = kept + act.astype(f32) * (df > tg).astype(f32)
            act = jnp.where(d_b == tg.astype(bf16), act, zb)

    kp = kept * p_all
    z = jnp.zeros((), f32)
    erh = erow[:, :T]
    dispT = (jnp.where(erh == e_all[:, :T], kept[:, :T], z) +
             jnp.where(erh == e_all[:, T:], kept[:, T:], z))
    combT = (jnp.where(erh == e_all[:, :T], kp[:, :T], z) +
             jnp.where(erh == e_all[:, T:], kp[:, T:], z))
    ii = lax.broadcasted_iota(jnp.int32, (E, E), 0)
    jj = lax.broadcasted_iota(jnp.int32, (E, E), 1)
    ident = (ii == jj).astype(f32)
    disp_ref[...] = lax.dot_general(dispT, ident, (((0,), (0,)), ((), ())),
                                    preferred_element_type=f32)
    comb_ref[...] = lax.dot_general(combT, ident, (((0,), (0,)), ((), ())),
                                    preferred_element_type=f32)
    rppe = psum / f32(T)
    usage = counts / f32(T2)
    aux_ref[...] = jnp.sum(rppe * usage, axis=1, keepdims=True) * f32(E)


def _body(x_ref, w_ref, probs_ref, disp_ref, comb_ref, aux_ref,
          ep_s, psum_s, id_s, *, n_tokens, n_experts, capacity, tok_blk,
          n_blk):
    i = pl.program_id(0)
    f32 = jnp.float32

    @pl.when(i == 0)
    def _():
        psum_s[...] = jnp.zeros_like(psum_s)
        ri = lax.broadcasted_iota(jnp.int32, (tok_blk, tok_blk), 0)
        rj = lax.broadcasted_iota(jnp.int32, (tok_blk, tok_blk), 1)
        id_s[...] = (ri == rj).astype(f32)

    x = x_ref[...]
    w = w_ref[...]
    logits = lax.dot_general(x, w, (((1,), (1,)), ((), ())),
                             preferred_element_type=f32)
    m = jnp.max(logits, axis=1, keepdims=True)
    ex = jnp.exp(logits - m)
    s = jnp.sum(ex, axis=1, keepdims=True)
    probs = ex / s
    probs_ref[...] = probs
    psum_s[...] += jnp.sum(probs, axis=0, keepdims=True)

    lane = lax.broadcasted_iota(jnp.int32, probs.shape, 1)
    m1 = jnp.max(probs, axis=1, keepdims=True)
    e1 = jnp.min(jnp.where(probs == m1, lane, n_experts), axis=1,
                 keepdims=True)
    pm = jnp.where(lane == e1, -1.0, probs)
    m2 = jnp.max(pm, axis=1, keepdims=True)
    e2 = jnp.min(jnp.where(pm == m2, lane, n_experts), axis=1, keepdims=True)
    denom = m1 + m2
    cols = jnp.concatenate(
        [e1.astype(f32), e2.astype(f32), m1 / denom, m2 / denom], axis=1)
    rows4 = lax.dot_general(cols, id_s[...], (((0,), (0,)), ((), ())),
                            preferred_element_type=f32)
    ep_s[:, pl.ds(i * tok_blk, tok_blk)] = rows4

    @pl.when(i == n_blk - 1)
    def _():
        _select_and_emit(ep_s[...], psum_s[...], disp_ref, comb_ref, aux_ref,
                         n_tokens=n_tokens, n_experts=n_experts,
                         capacity=capacity)


def kernel(x, W):
    batch, seq, d_model = x.shape
    n_experts = W.shape[0]
    n_tokens = batch * seq
    capacity = int(n_tokens * CAPACITY_FACTOR * 2 / n_experts)
    tok_blk = min(TOK_BLK, n_tokens)
    n_blk = n_tokens // tok_blk
    f32 = jnp.float32

    x2 = x.reshape(n_tokens, d_model)
    out = pl.pallas_call(
        functools.partial(_body, n_tokens=n_tokens, n_experts=n_experts,
                          capacity=capacity, tok_blk=tok_blk, n_blk=n_blk),
        grid=(n_blk,),
        in_specs=[
            pl.BlockSpec((tok_blk, d_model), lambda i: (i, 0)),
            pl.BlockSpec((n_experts, d_model), lambda i: (0, 0)),
        ],
        out_specs=[
            pl.BlockSpec((tok_blk, n_experts), lambda i: (i, 0)),
            pl.BlockSpec((n_tokens, n_experts), lambda i: (0, 0)),
            pl.BlockSpec((n_tokens, n_experts), lambda i: (0, 0)),
            pl.BlockSpec((1, 1), lambda i: (0, 0)),
        ],
        out_shape=[
            jax.ShapeDtypeStruct((n_tokens, n_experts), f32),
            jax.ShapeDtypeStruct((n_tokens, n_experts), f32),
            jax.ShapeDtypeStruct((n_tokens, n_experts), f32),
            jax.ShapeDtypeStruct((1, 1), f32),
        ],
        scratch_shapes=[
            pltpu.VMEM((4, n_tokens), f32),
            pltpu.VMEM((1, n_experts), f32),
            pltpu.VMEM((tok_blk, tok_blk), f32),
        ],
    )
    probs, dispatch, combine, aux = out(x2, W)

    return (dispatch.reshape(batch, seq, n_experts),
            combine.reshape(batch, seq, n_experts),
            probs.reshape(batch, seq, n_experts),
            aux.reshape(()))


# speedup vs baseline: 1.0534x; 1.0534x over previous
import functools

import jax
import jax.numpy as jnp
from jax import lax
from jax.experimental import pallas as pl
from jax.experimental.pallas import tpu as pltpu

CAPACITY_FACTOR = 1.1
TOK_BLK = 512


def _select_and_emit(e_all, p_all, psum, disp_ref, comb_ref, aux_ref, *,
                     n_tokens, n_experts, capacity):
    f32 = jnp.float32
    T = n_tokens
    T2 = 2 * n_tokens
    E = n_experts
    nb = 64

    hi = lax.bitcast_convert_type(p_all, jnp.int32)
    pos = lax.broadcasted_iota(jnp.int32, (1, T2), 1)
    flat = jnp.where(pos < T, 2 * pos, 2 * (pos - T) + 1)
    lo = (T2 - 1) - flat

    erow = lax.broadcasted_iota(jnp.int32, (E, T2), 0)
    oh = (erow == e_all).astype(f32)

    kept = jnp.zeros((1, T2), f32)
    act = jnp.ones((1, T2), f32)
    need = jnp.full((1, E), capacity, f32)

    bv = lax.broadcasted_iota(jnp.int32, (nb, nb), 0)
    bu = lax.broadcasted_iota(jnp.int32, (nb, nb), 1)
    cumM = (bu >= bv).astype(f32)
    vio0 = lax.broadcasted_iota(jnp.int32, (nb, E), 0)
    drow = lax.broadcasted_iota(jnp.int32, (nb, T2), 0)

    counts = None
    passes = [(hi, sh) for sh in (24, 18, 12, 6, 0)] + \
             [(lo, sh) for sh in (12, 6, 0)]
    n_pass = len(passes)
    for pi, (w_, sh) in enumerate(passes):
        d = (w_ >> sh) & (nb - 1)
        df = d.astype(f32)
        Dp = (drow == d).astype(f32) * act
        hist = lax.dot_general(Dp, oh, (((1,), (1,)), ((), ())),
                               preferred_element_type=f32)
        cum = lax.dot_general(cumM, hist, (((1,), (0,)), ((), ())),
                              preferred_element_type=f32)
        if pi == 0:
            counts = cum[0:1, :]
        t_row = jnp.max(jnp.where(cum >= need, vio0, -1), axis=0,
                        keepdims=True)
        cnt = jnp.sum(jnp.where(vio0 == t_row, cum - hist, 0.0), axis=0,
                      keepdims=True)
        cnt = jnp.where(t_row == -1, cum[0:1, :], cnt)
        need = need - cnt
        tg = lax.dot_general(t_row.astype(f32), oh, (((1,), (0,)), ((), ())),
                             preferred_element_type=f32)
        if pi == n_pass - 1:
            kept = kept + act * (df >= tg).astype(f32)
        else:
            kept = kept + act * (df > tg).astype(f32)
            act = act * (df == tg).astype(f32)

    kp = kept * p_all
    dispT = oh[:, :T] * kept[:, :T] + oh[:, T:] * kept[:, T:]
    combT = oh[:, :T] * kp[:, :T] + oh[:, T:] * kp[:, T:]
    ii = lax.broadcasted_iota(jnp.int32, (E, E), 0)
    jj = lax.broadcasted_iota(jnp.int32, (E, E), 1)
    ident = (ii == jj).astype(f32)
    disp_ref[...] = lax.dot_general(dispT, ident, (((0,), (0,)), ((), ())),
                                    preferred_element_type=f32)
    comb_ref[...] = lax.dot_general(combT, ident, (((0,), (0,)), ((), ())),
                                    preferred_element_type=f32)
    rppe = psum / f32(T)
    usage = counts / f32(T2)
    aux_ref[...] = jnp.sum(rppe * usage, axis=1, keepdims=True) * f32(E)


def _body(x_ref, w_ref, probs_ref, disp_ref, comb_ref, aux_ref,
          e_s, p_s, psum_s, *, n_tokens, n_experts, capacity, tok_blk,
          n_blk):
    i = pl.program_id(0)
    x = x_ref[...]
    w = w_ref[...]
    logits = lax.dot_general(x, w, (((1,), (1,)), ((), ())),
                             preferred_element_type=jnp.float32)
    m = jnp.max(logits, axis=1, keepdims=True)
    ex = jnp.exp(logits - m)
    s = jnp.sum(ex, axis=1, keepdims=True)
    probs = ex / s
    probs_ref[...] = probs

    @pl.when(i == 0)
    def _():
        psum_s[...] = jnp.zeros_like(psum_s)

    psum_s[...] += jnp.sum(probs, axis=0, keepdims=True)

    pT = jnp.transpose(probs)
    lane0 = lax.broadcasted_iota(jnp.int32, pT.shape, 0)
    m1 = jnp.max(pT, axis=0, keepdims=True)
    e1 = jnp.min(jnp.where(pT == m1, lane0, n_experts), axis=0,
                 keepdims=True)
    pm = jnp.where(lane0 == e1, -1.0, pT)
    m2 = jnp.max(pm, axis=0, keepdims=True)
    e2 = jnp.min(jnp.where(pm == m2, lane0, n_experts), axis=0, keepdims=True)
    denom = m1 + m2
    e_s[0:1, pl.ds(i * tok_blk, tok_blk)] = e1
    e_s[0:1, pl.ds(n_tokens + i * tok_blk, tok_blk)] = e2
    p_s[0:1, pl.ds(i * tok_blk, tok_blk)] = m1 / denom
    p_s[0:1, pl.ds(n_tokens + i * tok_blk, tok_blk)] = m2 / denom

    @pl.when(i == n_blk - 1)
    def _():
        _select_and_emit(e_s[...], p_s[...], psum_s[...],
                         disp_ref, comb_ref, aux_ref,
                         n_tokens=n_tokens, n_experts=n_experts,
                         capacity=capacity)


def kernel(x, W):
    batch, seq, d_model = x.shape
    n_experts = W.shape[0]
    n_tokens = batch * seq
    capacity = int(n_tokens * CAPACITY_FACTOR * 2 / n_experts)
    tok_blk = min(TOK_BLK, n_tokens)
    n_blk = n_tokens // tok_blk
    f32 = jnp.float32

    x2 = x.reshape(n_tokens, d_model)
    out = pl.pallas_call(
        functools.partial(_body, n_tokens=n_tokens, n_experts=n_experts,
                          capacity=capacity, tok_blk=tok_blk, n_blk=n_blk),
        grid=(n_blk,),
        in_specs=[
            pl.BlockSpec((tok_blk, d_model), lambda i: (i, 0)),
            pl.BlockSpec((n_experts, d_model), lambda i: (0, 0)),
        ],
        out_specs=[
            pl.BlockSpec((tok_blk, n_experts), lambda i: (i, 0)),
            pl.BlockSpec((n_tokens, n_experts), lambda i: (0, 0)),
            pl.BlockSpec((n_tokens, n_experts), lambda i: (0, 0)),
            pl.BlockSpec((1, 1), lambda i: (0, 0)),
        ],
        out_shape=[
            jax.ShapeDtypeStruct((n_tokens, n_experts), f32),
            jax.ShapeDtypeStruct((n_tokens, n_experts), f32),
            jax.ShapeDtypeStruct((n_tokens, n_experts), f32),
            jax.ShapeDtypeStruct((1, 1), f32),
        ],
        scratch_shapes=[
            pltpu.VMEM((1, 2 * n_tokens), jnp.int32),
            pltpu.VMEM((1, 2 * n_tokens), f32),
            pltpu.VMEM((1, n_experts), f32),
        ],
    )
    probs, dispatch, combine, aux = out(x2, W)

    return (dispatch.reshape(batch, seq, n_experts),
            combine.reshape(batch, seq, n_experts),
            probs.reshape(batch, seq, n_experts),
            aux.reshape(()))
